# baseline (speedup 1.0000x reference)
import functools

import jax
import jax.numpy as jnp
from jax import lax
from jax.experimental import pallas as pl
from jax.experimental.pallas import tpu as pltpu
from jax.experimental.pallas import tpu_sc as plsc


def _sc_gather_body(NC, NCH, CH, HP, n_per_w, NBUF,
                    h_hbm, c_hbm, idx0_hbm, idx1_hbm, out_h, out_c,
                    i0_v, i1_v, h0_v, h1_v, c0_v, c1_v, sem0, sem1):
    wid = lax.axis_index("s") * NC + lax.axis_index("c")
    sems = (sem0, sem1)

    def gathers(b, k):
        base = wid * n_per_w + k * CH
        return (
            pltpu.make_async_copy(idx0_hbm.at[pl.ds(base, CH)], i0_v.at[b],
                                  sems[b]),
            pltpu.make_async_copy(idx1_hbm.at[pl.ds(base, CH)], i1_v.at[b],
                                  sems[b]),
            pltpu.make_async_copy(h_hbm.at[i0_v.at[b]], h0_v.at[b], sems[b]),
            pltpu.make_async_copy(h_hbm.at[i1_v.at[b]], h1_v.at[b], sems[b]),
            pltpu.make_async_copy(c_hbm.at[i0_v.at[b]], c0_v.at[b], sems[b]),
            pltpu.make_async_copy(c_hbm.at[i1_v.at[b]], c1_v.at[b], sems[b]),
        )

    def fire(b, k):
        cps = gathers(b, k)
        cps[0].start()
        cps[1].start()
        cps[0].wait()
        cps[1].wait()
        for cp in cps[2:]:
            cp.start()

    def drain_write(b, k):
        base = wid * n_per_w + k * CH
        for cp in gathers(b, k)[2:]:
            cp.wait()
        pltpu.sync_copy(h0_v.at[b], out_h.at[pl.ds(base, CH), pl.ds(0, HP)])
        pltpu.sync_copy(h1_v.at[b], out_h.at[pl.ds(base, CH), pl.ds(HP, HP)])
        pltpu.sync_copy(c0_v.at[b], out_c.at[pl.ds(base, CH), pl.ds(0, HP)])
        pltpu.sync_copy(c1_v.at[b], out_c.at[pl.ds(base, CH), pl.ds(HP, HP)])

    for b in range(NBUF):
        fire(b, b)

    def body(it, carry):
        g = it * NBUF
        for b in range(NBUF):
            k = g + b
            drain_write(b, k)

            @pl.when(k + NBUF < NCH)
            def _():
                fire(b, k + NBUF)
        return carry

    lax.fori_loop(0, NCH // NBUF, body, 0)


def _make_sc_gather(n_pad, HP):
    info = plsc.get_sparse_core_info()
    NC, NS = info.num_cores, info.num_subcores
    NW = NC * NS
    assert n_pad % NW == 0
    n_per_w = n_pad // NW
    CH = 80
    NBUF = 2
    assert n_per_w % CH == 0
    NCH = n_per_w // CH
    assert NCH % NBUF == 0

    mesh = plsc.VectorSubcoreMesh(core_axis_name="c", subcore_axis_name="s")
    return functools.partial(
        pl.kernel,
        functools.partial(_sc_gather_body, NC, NCH, CH, HP, n_per_w, NBUF),
        out_type=[jax.ShapeDtypeStruct((n_pad, 2 * HP), jnp.float32),
                  jax.ShapeDtypeStruct((n_pad, 2 * HP), jnp.float32)],
        mesh=mesh,
        scratch_types=[
            pltpu.VMEM((NBUF, CH), jnp.int32),
            pltpu.VMEM((NBUF, CH), jnp.int32),
            pltpu.VMEM((NBUF, CH, HP), jnp.float32),
            pltpu.VMEM((NBUF, CH, HP), jnp.float32),
            pltpu.VMEM((NBUF, CH, HP), jnp.float32),
            pltpu.VMEM((NBUF, CH, HP), jnp.float32),
            pltpu.SemaphoreType.DMA,
            pltpu.SemaphoreType.DMA,
        ],
    )()


def _unpack(x):
    w = lax.bitcast_convert_type(x, jnp.uint32)
    lo = lax.bitcast_convert_type(w << jnp.uint32(16), jnp.float32)
    hi = lax.bitcast_convert_type(w & jnp.uint32(0xFFFF0000), jnp.float32)
    return lo, hi


def _tc_body(HH, hcat_ref, cc_ref, wft_ref, bf_ref, wiout_ref, biou_ref,
             hnew_ref, cnew_ref):
    HP = HH // 2
    pk = hcat_ref[...]
    lo0, hi0 = _unpack(pk[:, :HP])
    lo1, hi1 = _unpack(pk[:, HP:])
    hcat = jnp.concatenate([lo0, hi0, lo1, hi1],
                           axis=1).astype(jnp.bfloat16)
    f_pre = jnp.dot(hcat, wft_ref[...],
                    preferred_element_type=jnp.float32) + bf_ref[...]
    f = jax.nn.sigmoid(f_pre)
    ck = cc_ref[...]
    clo0, chi0 = _unpack(ck[:, :HP])
    clo1, chi1 = _unpack(ck[:, HP:])
    cc = jnp.concatenate([clo0, chi0, clo1, chi1], axis=1)
    c_red = f[:, :HH] * cc[:, :HH] + f[:, HH:] * cc[:, HH:]
    iou = jnp.dot(hcat, wiout_ref[...],
                  preferred_element_type=jnp.float32) + biou_ref[...]
    i = jax.nn.sigmoid(iou[:, :HH])
    o = jax.nn.sigmoid(iou[:, HH:2 * HH])
    u = jnp.tanh(iou[:, 2 * HH:])
    c_new = i * u + c_red
    hnew_ref[...] = o * jnp.tanh(c_new)
    cnew_ref[...] = c_new


def _tc_compute(n, hcat, cc, wft, bf, wiout, biou, M=2000):
    twoHP = hcat.shape[1]
    twoH = 2 * twoHP
    HH = twoH // 2
    grid = (n // M,)
    return pl.pallas_call(
        functools.partial(_tc_body, HH),
        grid=grid,
        in_specs=[
            pl.BlockSpec((M, twoHP), lambda i: (i, 0)),
            pl.BlockSpec((M, twoHP), lambda i: (i, 0)),
            pl.BlockSpec((twoH, twoH), lambda i: (0, 0)),
            pl.BlockSpec((1, twoH), lambda i: (0, 0)),
            pl.BlockSpec((twoH, 3 * HH), lambda i: (0, 0)),
            pl.BlockSpec((1, 3 * HH), lambda i: (0, 0)),
        ],
        out_specs=[
            pl.BlockSpec((M, HH), lambda i: (i, 0)),
            pl.BlockSpec((M, HH), lambda i: (i, 0)),
        ],
        out_shape=[
            jax.ShapeDtypeStruct((n, HH), jnp.float32),
            jax.ShapeDtypeStruct((n, HH), jnp.float32),
        ],
    )(hcat, cc, wft, bf, wiout, biou)


def _pack_bf16(x):
    k = x.shape[1] // 2
    st = jnp.stack([x[:, :k], x[:, k:]], axis=-1).astype(jnp.bfloat16)
    return lax.bitcast_convert_type(st, jnp.float32)


def kernel(h, c, child_idx, W_f, b_f, W_iou, b_iou):
    n, HH = h.shape
    HP = HH // 2
    NW = 32
    n_per_w = -(-n // (NW * 160)) * 160
    n_pad = n_per_w * NW

    ci = child_idx.astype(jnp.int32)
    pad = jnp.zeros((n_pad - n,), jnp.int32)
    idx0 = jnp.concatenate([ci[:, 0], pad])
    idx1 = jnp.concatenate([ci[:, 1], pad])

    sc_gather = _make_sc_gather(n_pad, HP)
    hcat, ccat = sc_gather(_pack_bf16(h), _pack_bf16(c), idx0, idx1)

    h_new, c_new = _tc_compute(
        n, hcat, ccat,
        W_f.T.astype(jnp.bfloat16), b_f.reshape(1, -1),
        W_iou.T.astype(jnp.bfloat16), b_iou.reshape(1, -1))
    return (h_new, c_new)


# speedup vs baseline: 1.4132x; 1.4132x over previous
import functools

import jax
import jax.numpy as jnp
from jax import lax
from jax.experimental import pallas as pl
from jax.experimental.pallas import tpu as pltpu
from jax.experimental.pallas import tpu_sc as plsc


def _rne16(u):
    return (u + jnp.uint32(0x7FFF) + ((u >> jnp.uint32(16)) & jnp.uint32(1))
            ) >> jnp.uint32(16)


def _pack_pair(x, HP):
    a = lax.bitcast_convert_type(x[:, :HP], jnp.uint32)
    b = lax.bitcast_convert_type(x[:, HP:], jnp.uint32)
    w = (_rne16(a) & jnp.uint32(0xFFFF)) | (_rne16(b) << jnp.uint32(16))
    return lax.bitcast_convert_type(w, jnp.float32)


def _pack_body(HP, h_ref, c_ref, hpk_ref, cpk_ref):
    hpk_ref[...] = _pack_pair(h_ref[...], HP)
    cpk_ref[...] = _pack_pair(c_ref[...], HP)


def _tc_pack(h, c, Mp=2000):
    n, HH = h.shape
    HP = HH // 2
    return pl.pallas_call(
        functools.partial(_pack_body, HP),
        grid=(n // Mp,),
        in_specs=[
            pl.BlockSpec((Mp, HH), lambda i: (i, 0)),
            pl.BlockSpec((Mp, HH), lambda i: (i, 0)),
        ],
        out_specs=[
            pl.BlockSpec((Mp, HP), lambda i: (i, 0)),
            pl.BlockSpec((Mp, HP), lambda i: (i, 0)),
        ],
        out_shape=[
            jax.ShapeDtypeStruct((n, HP), jnp.float32),
            jax.ShapeDtypeStruct((n, HP), jnp.float32),
        ],
    )(h, c)


def _sc_gather_body(NC, NCH, CH, HP, n_per_w, NBUF, DEPTH,
                    h_hbm, c_hbm, idx0_hbm, idx1_hbm, out_h, out_c,
                    i0_all, i1_all, h0_v, h1_v, c0_v, c1_v, *sems):
    gsems = sems[:NBUF]
    wsems = sems[NBUF:]
    wid = lax.axis_index("s") * NC + lax.axis_index("c")
    wbase = wid * n_per_w
    pltpu.sync_copy(idx0_hbm.at[pl.ds(wbase, n_per_w)], i0_all)
    pltpu.sync_copy(idx1_hbm.at[pl.ds(wbase, n_per_w)], i1_all)

    def g_cps(b, k):
        io0 = i0_all.at[pl.ds(k * CH, CH)]
        io1 = i1_all.at[pl.ds(k * CH, CH)]
        return (
            pltpu.make_async_copy(h_hbm.at[io0], h0_v.at[b], gsems[b]),
            pltpu.make_async_copy(h_hbm.at[io1], h1_v.at[b], gsems[b]),
            pltpu.make_async_copy(c_hbm.at[io0], c0_v.at[b], gsems[b]),
            pltpu.make_async_copy(c_hbm.at[io1], c1_v.at[b], gsems[b]),
        )

    def w_cps(b, k):
        rows = pl.ds(wbase + k * CH, CH)
        return (
            pltpu.make_async_copy(h0_v.at[b], out_h.at[rows, pl.ds(0, HP)],
                                  wsems[b]),
            pltpu.make_async_copy(h1_v.at[b], out_h.at[rows, pl.ds(HP, HP)],
                                  wsems[b]),
            pltpu.make_async_copy(c0_v.at[b], out_c.at[rows, pl.ds(0, HP)],
                                  wsems[b]),
            pltpu.make_async_copy(c1_v.at[b], out_c.at[rows, pl.ds(HP, HP)],
                                  wsems[b]),
        )

    def body(it, carry):
        g = it * NBUF
        for b in range(NBUF):
            k = g + b

            @pl.when(k >= NBUF)
            def _():
                for cp in w_cps(b, k - NBUF):
                    cp.wait()

            @pl.when(k < NCH)
            def _():
                for cp in g_cps(b, k):
                    cp.start()

            kd = k - DEPTH
            bd = (b - DEPTH) % NBUF

            @pl.when((kd >= 0) & (kd < NCH))
            def _():
                for cp in g_cps(bd, kd):
                    cp.wait()
                for cp in w_cps(bd, kd):
                    cp.start()
        return carry

    lax.fori_loop(0, (NCH + NBUF) // NBUF, body, 0)


def _make_sc_gather(n_pad, HP):
    info = plsc.get_sparse_core_info()
    NC, NS = info.num_cores, info.num_subcores
    NW = NC * NS
    assert n_pad % NW == 0
    n_per_w = n_pad // NW
    CH = 40
    NBUF = 4
    DEPTH = 2
    assert n_per_w % CH == 0
    NCH = n_per_w // CH
    assert NCH % NBUF == 0

    mesh = plsc.VectorSubcoreMesh(core_axis_name="c", subcore_axis_name="s")
    return functools.partial(
        pl.kernel,
        functools.partial(_sc_gather_body, NC, NCH, CH, HP, n_per_w, NBUF,
                          DEPTH),
        out_type=[jax.ShapeDtypeStruct((n_pad, 2 * HP), jnp.float32),
                  jax.ShapeDtypeStruct((n_pad, 2 * HP), jnp.float32)],
        mesh=mesh,
        scratch_types=(
            [pltpu.VMEM((n_per_w,), jnp.int32),
             pltpu.VMEM((n_per_w,), jnp.int32)] +
            [pltpu.VMEM((NBUF, CH, HP), jnp.float32) for _ in range(4)] +
            [pltpu.SemaphoreType.DMA for _ in range(2 * NBUF)]
        ),
    )()


def _unpack(x):
    w = lax.bitcast_convert_type(x, jnp.uint32)
    lo = lax.bitcast_convert_type(w << jnp.uint32(16), jnp.float32)
    hi = lax.bitcast_convert_type(w & jnp.uint32(0xFFFF0000), jnp.float32)
    return lo, hi


def _tc_body(HH, hcat_ref, cc_ref, wft_ref, bf_ref, wiout_ref, biou_ref,
             hnew_ref, cnew_ref):
    HP = HH // 2
    pk = hcat_ref[...]
    lo0, hi0 = _unpack(pk[:, :HP])
    lo1, hi1 = _unpack(pk[:, HP:])
    hcat = jnp.concatenate([lo0, hi0, lo1, hi1],
                           axis=1).astype(jnp.bfloat16)
    f_pre = jnp.dot(hcat, wft_ref[...],
                    preferred_element_type=jnp.float32) + bf_ref[...]
    f = jax.nn.sigmoid(f_pre)
    ck = cc_ref[...]
    clo0, chi0 = _unpack(ck[:, :HP])
    clo1, chi1 = _unpack(ck[:, HP:])
    cc = jnp.concatenate([clo0, chi0, clo1, chi1], axis=1)
    c_red = f[:, :HH] * cc[:, :HH] + f[:, HH:] * cc[:, HH:]
    iou = jnp.dot(hcat, wiout_ref[...],
                  preferred_element_type=jnp.float32) + biou_ref[...]
    i = jax.nn.sigmoid(iou[:, :HH])
    o = jax.nn.sigmoid(iou[:, HH:2 * HH])
    u = jnp.tanh(iou[:, 2 * HH:])
    c_new = i * u + c_red
    hnew_ref[...] = o * jnp.tanh(c_new)
    cnew_ref[...] = c_new


def _tc_compute(n, hcat, cc, wft, bf, wiout, biou, M=2000):
    twoHP = hcat.shape[1]
    twoH = 2 * twoHP
    HH = twoH // 2
    grid = (n // M,)
    return pl.pallas_call(
        functools.partial(_tc_body, HH),
        grid=grid,
        in_specs=[
            pl.BlockSpec((M, twoHP), lambda i: (i, 0)),
            pl.BlockSpec((M, twoHP), lambda i: (i, 0)),
            pl.BlockSpec((twoH, twoH), lambda i: (0, 0)),
            pl.BlockSpec((1, twoH), lambda i: (0, 0)),
            pl.BlockSpec((twoH, 3 * HH), lambda i: (0, 0)),
            pl.BlockSpec((1, 3 * HH), lambda i: (0, 0)),
        ],
        out_specs=[
            pl.BlockSpec((M, HH), lambda i: (i, 0)),
            pl.BlockSpec((M, HH), lambda i: (i, 0)),
        ],
        out_shape=[
            jax.ShapeDtypeStruct((n, HH), jnp.float32),
            jax.ShapeDtypeStruct((n, HH), jnp.float32),
        ],
    )(hcat, cc, wft, bf, wiout, biou)


def kernel(h, c, child_idx, W_f, b_f, W_iou, b_iou):
    n, HH = h.shape
    HP = HH // 2
    NW = 32
    n_per_w = -(-n // (NW * 160)) * 160
    n_pad = n_per_w * NW

    ci = child_idx.astype(jnp.int32)
    pad = jnp.zeros((n_pad - n,), jnp.int32)
    idx0 = jnp.concatenate([ci[:, 0], pad])
    idx1 = jnp.concatenate([ci[:, 1], pad])

    h_pk, c_pk = _tc_pack(h, c)
    sc_gather = _make_sc_gather(n_pad, HP)
    hcat, ccat = sc_gather(h_pk, c_pk, idx0, idx1)

    h_new, c_new = _tc_compute(
        n, hcat, ccat,
        W_f.T.astype(jnp.bfloat16), b_f.reshape(1, -1),
        W_iou.T.astype(jnp.bfloat16), b_iou.reshape(1, -1))
    return (h_new, c_new)
